# baseline (speedup 1.0000x reference)
import jax
import jax.numpy as jnp
from jax import lax
from jax.experimental import pallas as pl
from jax.experimental.pallas import tpu as pltpu
from jax.experimental.pallas import tpu_sc as plsc

VOCAB = 1000000
D = 64
HIST = 50
BATCH = 4096
NUM_LABELS = 2

NC = 2
NS = 16
NW = NC * NS

BAGS_PER_W = BATCH // NW
PAIRS_PER_W = BAGS_PER_W // 2
ROWS_PER_CHUNK = 2 * HIST


def _sc_bag_sums(feat_hbm, table_hbm, out_hbm, idx_v, rows_v, sums_v, sem):
    wid = lax.axis_index("s") * NC + lax.axis_index("c")
    pair_base = wid * PAIRS_PER_W

    pltpu.sync_copy(feat_hbm.at[pl.ds(pair_base, PAIRS_PER_W)], idx_v)

    zero = jnp.zeros((16,), jnp.float32)

    def do_chunk(j, _):
        pltpu.async_copy(table_hbm.at[idx_v.at[j]], rows_v, sem).wait()

        for bag in range(2):
            def body(r, acc):
                a0, a1, a2, a3 = acc
                row = bag * HIST + r
                a0 = a0 + rows_v[row, pl.ds(0, 16)]
                a1 = a1 + rows_v[row, pl.ds(16, 16)]
                a2 = a2 + rows_v[row, pl.ds(32, 16)]
                a3 = a3 + rows_v[row, pl.ds(48, 16)]
                return (a0, a1, a2, a3)

            a0, a1, a2, a3 = lax.fori_loop(0, HIST, body,
                                           (zero, zero, zero, zero))
            bag_idx = 2 * j + bag
            sums_v[bag_idx, pl.ds(0, 16)] = a0
            sums_v[bag_idx, pl.ds(16, 16)] = a1
            sums_v[bag_idx, pl.ds(32, 16)] = a2
            sums_v[bag_idx, pl.ds(48, 16)] = a3
        return 0

    lax.fori_loop(0, PAIRS_PER_W, do_chunk, 0)

    pltpu.sync_copy(sums_v, out_hbm.at[pl.ds(wid * BAGS_PER_W, BAGS_PER_W)])


def _tc_linear(sums_ref, w_ref, b_ref, o_ref):
    x = sums_ref[...] * jnp.float32(1.0 / HIST)
    w = w_ref[...]
    o_ref[...] = (
        jax.lax.dot_general(x, w, (((1,), (1,)), ((), ())),
                            preferred_element_type=jnp.float32)
        + b_ref[...]
    )


@jax.jit
def _run(feat_pairs, emb_table, W, b):
    mesh = plsc.VectorSubcoreMesh(core_axis_name="c", subcore_axis_name="s",
                                  num_cores=NC, num_subcores=NS)
    sums = pl.kernel(
        _sc_bag_sums,
        out_type=jax.ShapeDtypeStruct((BATCH, D), jnp.float32),
        mesh=mesh,
        compiler_params=pltpu.CompilerParams(use_tc_tiling_on_sc=False),
        scratch_types=[
            pltpu.VMEM((PAIRS_PER_W, ROWS_PER_CHUNK), jnp.int32),
            pltpu.VMEM((ROWS_PER_CHUNK, D), jnp.float32),
            pltpu.VMEM((BAGS_PER_W, D), jnp.float32),
            pltpu.SemaphoreType.DMA,
        ],
    )(feat_pairs, emb_table)

    logits = pl.pallas_call(
        _tc_linear,
        out_shape=jax.ShapeDtypeStruct((BATCH, NUM_LABELS), jnp.float32),
    )(sums, W, b.reshape(1, NUM_LABELS))
    return logits


def kernel(features, emb_table, W, b):
    feat_pairs = features.astype(jnp.int32).reshape(BATCH // 2, 2 * HIST)
    return _run(feat_pairs, emb_table, W, b.astype(jnp.float32))
